# baseline (speedup 1.0000x reference)
import functools

import jax
import jax.numpy as jnp
from jax import lax
from jax.experimental import pallas as pl
from jax.experimental.pallas import tpu as pltpu
from jax.experimental.pallas import tpu_sc as plsc

N = 10000
E = 160000
DF = 128
DE = 16
H = 128
D = 32
B = 128
NCONV = 2
NS2S = 2

NC = 2
NS = 16
NW = NC * NS
CHUNK = 128
SUB = 5
KB = SUB * CHUNK
CPW = 8
EPW = CPW * KB
EHALF = NW * EPW
NHALF = 1
EPAD = NHALF * EHALF
NPAD = 10240
ZR = NPAD // NS

EB = 2048

_f32 = jnp.float32


def _proj_body(x_ref, w_ref, b_ref, o_ref):
    o_ref[...] = jnp.maximum(
        jnp.dot(x_ref[...], w_ref[...], preferred_element_type=_f32)
        + b_ref[...], 0.0)


def _proj(x, W0, b0):
    return pl.pallas_call(
        _proj_body,
        out_shape=jax.ShapeDtypeStruct((N, D), _f32),
    )(x, W0, b0.reshape(1, D))


def _msg_body(ea_ref, xs_ref, l1_ref, bn1_ref, l2_ref, bn2r_ref, r_ref, s_ref,
              msg_ref):
    hb = EB // 2
    for p in range(2):
        rows = pl.ds(p * hb, hb)
        h1 = jnp.maximum(
            jnp.dot(ea_ref[rows, :], l1_ref[...], preferred_element_type=_f32)
            + bn1_ref[...], 0.0)
        ew = jnp.dot(h1, l2_ref[...], preferred_element_type=_f32)
        xs = xs_ref[rows, :]
        xr = jnp.dot(xs, r_ref[...], preferred_element_type=_f32)
        msg_ref[rows, :] = (
            jnp.dot(ew * xr, s_ref[...], preferred_element_type=_f32)
            + jnp.dot(xs, bn2r_ref[...], preferred_element_type=_f32))


def _msg(ea, xs, L1, bn1, L2, Bn2r, Rm, Sm, blk0):
    grid = (EHALF // EB,)
    nblk = (E - 1) // EB
    full = lambda shape: pl.BlockSpec(shape, lambda i: (0, 0))
    return pl.pallas_call(
        _msg_body,
        grid=grid,
        in_specs=[
            pl.BlockSpec((EB, DE), lambda i: (jnp.minimum(i + blk0, nblk), 0)),
            pl.BlockSpec((EB, D), lambda i: (i, 0)),
            full((DE, H)),
            full((1, H)),
            pl.BlockSpec((H, D * D), lambda i: (0, 0)),
            full((D, D)),
            full((D, D * D)),
            full((D * D, D)),
        ],
        out_specs=pl.BlockSpec((EB, D), lambda i: (i, 0)),
        out_shape=jax.ShapeDtypeStruct((EHALF, D), _f32),
    )(ea, xs, L1, bn1.reshape(1, H), L2, Bn2r, Rm, Sm)


NB = 2000


def _gru_body(*refs):
    agg_refs = refs[0:NHALF]
    deg_refs = refs[NHALF:2 * NHALF]
    (out_ref, wroot_ref, cb_ref, wih_ref, whh_ref, bih_ref, bhh_ref,
     new_ref) = refs[2 * NHALF:]
    d = jnp.maximum(
        sum(r[c, :, 0:1] for r in deg_refs for c in range(NC)), 1.0)
    agg = sum(r[c, :, :] for r in agg_refs for c in range(NC)) / d
    out = out_ref[...]
    m = jnp.maximum(
        agg + jnp.dot(out, wroot_ref[...], preferred_element_type=_f32)
        + cb_ref[...], 0.0)
    gi = jnp.dot(m, wih_ref[...], preferred_element_type=_f32) + bih_ref[...]
    gh = jnp.dot(out, whh_ref[...], preferred_element_type=_f32) + bhh_ref[...]
    r = jax.nn.sigmoid(gi[:, 0:D] + gh[:, 0:D])
    z = jax.nn.sigmoid(gi[:, D:2 * D] + gh[:, D:2 * D])
    n = jnp.tanh(gi[:, 2 * D:3 * D] + r * gh[:, 2 * D:3 * D])
    new_ref[...] = (1.0 - z) * n + z * out


def _gru(aggs, degs, out, Wroot, cb, WihT, WhhT, bih, bhh):
    full = lambda shape: pl.BlockSpec(shape, lambda i: tuple(0 for _ in shape))
    return pl.pallas_call(
        _gru_body,
        grid=(N // NB,),
        in_specs=(
            [pl.BlockSpec((NC, NB, D), lambda i: (0, i, 0))] * NHALF
            + [pl.BlockSpec((NC, NB, 16), lambda i: (0, i, 0))] * NHALF
            + [
                pl.BlockSpec((NB, D), lambda i: (i, 0)),
                full((D, D)),
                full((1, D)),
                full((D, 3 * D)),
                full((D, 3 * D)),
                full((1, 3 * D)),
                full((1, 3 * D)),
            ]),
        out_specs=pl.BlockSpec((NB, D), lambda i: (i, 0)),
        out_shape=jax.ShapeDtypeStruct((N, D), _f32),
    )(*aggs, *degs, out, Wroot, cb.reshape(1, D), WihT, WhhT,
      bih.reshape(1, 3 * D), bhh.reshape(1, 3 * D))


def _s2s_body(out_ref, bcol_ref, wih_ref, whh_ref, bias_ref, wl1_ref, bl1_ref,
              wl2_ref, bl2_ref, o2_ref):
    out = out_ref[...]
    iota_row = lax.broadcasted_iota(jnp.int32, (1, B), 1)
    oh = (bcol_ref[...] == iota_row).astype(_f32)
    hs = jnp.zeros((B, D), _f32)
    cs = jnp.zeros((B, D), _f32)
    q_star = jnp.zeros((B, 2 * D), _f32)
    for _ in range(NS2S):
        gates = (jnp.dot(q_star, wih_ref[...], preferred_element_type=_f32)
                 + jnp.dot(hs, whh_ref[...], preferred_element_type=_f32)
                 + bias_ref[...])
        i_ = jax.nn.sigmoid(gates[:, 0:D])
        f_ = jax.nn.sigmoid(gates[:, D:2 * D])
        g_ = jnp.tanh(gates[:, 2 * D:3 * D])
        o_ = jax.nn.sigmoid(gates[:, 3 * D:4 * D])
        cs = f_ * cs + i_ * g_
        hs = o_ * jnp.tanh(cs)
        qb = jnp.dot(oh, hs, preferred_element_type=_f32)
        e = jnp.sum(out * qb, axis=1, keepdims=True)
        mx = jnp.max(jnp.where(oh > 0.0, e, -1e30), axis=0, keepdims=True)
        mxb = jnp.max(jnp.where(oh > 0.0, mx, -1e30), axis=1, keepdims=True)
        ex = jnp.exp(e - mxb)
        den = jnp.sum(oh * ex, axis=0, keepdims=True)
        denb = jnp.sum(jnp.where(oh > 0.0, den, 0.0), axis=1, keepdims=True)
        a = ex / (denb + 1e-16)
        rvec = lax.dot_general(oh, a * out, (((0,), (0,)), ((), ())),
                               preferred_element_type=_f32)
        q_star = jnp.concatenate([hs, rvec], axis=1)
    o1 = jnp.maximum(
        jnp.dot(q_star, wl1_ref[...], preferred_element_type=_f32)
        + bl1_ref[...], 0.0)
    o2_ref[...] = (jnp.dot(o1, wl2_ref[...], preferred_element_type=_f32)
                   + bl2_ref[...])


def _s2s(out, bcol, WihsT, WhhsT, bias_s, Wl1, bl1, Wl2, bl2):
    return pl.pallas_call(
        _s2s_body,
        out_shape=jax.ShapeDtypeStruct((B, 1), _f32),
    )(out, bcol, WihsT, WhhsT, bias_s, Wl1, bl1.reshape(1, D),
      Wl2, bl2.reshape(1, 1))


def _sc_gather_body(table_hbm, idx_hbm, xs_hbm, idx_v, rows_v, sem):
    c = lax.axis_index("c")
    s = lax.axis_index("s")
    wid = s * NC + c
    row0 = wid * (EPW // CHUNK)

    def body(k, carry):
        r0 = row0 + k * SUB
        pltpu.sync_copy(idx_hbm.at[pl.ds(r0, SUB)], idx_v)
        descs = []
        for j in range(SUB):
            descs.append(pltpu.async_copy(
                table_hbm.at[idx_v.at[j]],
                rows_v.at[pl.ds(j * CHUNK, CHUNK)], sem))
        for dsc in descs:
            dsc.wait()
        pltpu.sync_copy(rows_v, xs_hbm.at[pl.ds(r0 * CHUNK, KB)])
        return carry

    lax.fori_loop(0, CPW, body, 0)


def _sc_gather(out, src2d):
    mesh = plsc.VectorSubcoreMesh(core_axis_name="c", subcore_axis_name="s")
    return pl.kernel(
        _sc_gather_body,
        out_type=jax.ShapeDtypeStruct((EHALF, D), _f32),
        mesh=mesh,
        compiler_params=pltpu.CompilerParams(use_tc_tiling_on_sc=False),
        scratch_types=[
            pltpu.VMEM((SUB, CHUNK), jnp.int32),
            pltpu.VMEM((KB, D), _f32),
            pltpu.SemaphoreType.DMA,
        ],
    )(out, src2d)


ZB = ZR // CHUNK


def _sc_scatter_body(with_deg, *refs):
    if with_deg:
        (msg_hbm, idx_hbm, z32_hbm, z16_hbm, ones_hbm, agg_hbm, deg_hbm,
         aggbuf, degbuf, idx_v, rows_v, zc_v, z16c_v, ones_v, sem,
         semd) = refs
    else:
        (msg_hbm, idx_hbm, z32_hbm, agg_hbm,
         aggbuf, idx_v, rows_v, zc_v, sem) = refs
    c = lax.axis_index("c")
    s = lax.axis_index("s")
    wid = s * NC + c
    row0 = wid * (EPW // CHUNK)

    pltpu.sync_copy(z32_hbm, zc_v)
    for t in range(ZB):
        pltpu.sync_copy(zc_v, aggbuf.at[pl.ds(s * ZR + t * CHUNK, CHUNK)])
    if with_deg:
        pltpu.sync_copy(z16_hbm, z16c_v)
        pltpu.sync_copy(ones_hbm, ones_v)
        for t in range(ZB):
            pltpu.sync_copy(
                z16c_v, degbuf.at[pl.ds(s * ZR + t * CHUNK, CHUNK)])
    plsc.subcore_barrier()

    def body(k, carry):
        r0 = row0 + k * SUB
        pltpu.sync_copy(idx_hbm.at[pl.ds(r0, SUB)], idx_v)
        pltpu.sync_copy(msg_hbm.at[pl.ds(r0 * CHUNK, KB)], rows_v)
        descs = []
        for j in range(SUB):
            descs.append(pltpu.async_copy(
                rows_v.at[pl.ds(j * CHUNK, CHUNK)],
                aggbuf.at[idx_v.at[j]], sem, add=True))
            if with_deg:
                descs.append(pltpu.async_copy(
                    ones_v, degbuf.at[idx_v.at[j]], semd, add=True))
        for dsc in descs:
            dsc.wait()
        return carry

    lax.fori_loop(0, CPW, body, 0)
    plsc.subcore_barrier()

    pltpu.sync_copy(aggbuf.at[pl.ds(s * ZR, ZR)],
                    agg_hbm.at[c, pl.ds(s * ZR, ZR)])
    if with_deg:
        pltpu.sync_copy(degbuf.at[pl.ds(s * ZR, ZR)],
                        deg_hbm.at[c, pl.ds(s * ZR, ZR)])


def _sc_scatter(msg, dst2d, z32, z16=None, ones16=None, with_deg=True):
    mesh = plsc.VectorSubcoreMesh(core_axis_name="c", subcore_axis_name="s")
    if with_deg:
        return pl.kernel(
            functools.partial(_sc_scatter_body, True),
            out_type=[
                jax.ShapeDtypeStruct((NC, NPAD, D), _f32),
                jax.ShapeDtypeStruct((NC, NPAD, 16), _f32),
            ],
            mesh=mesh,
            compiler_params=pltpu.CompilerParams(use_tc_tiling_on_sc=False),
            scratch_types=[
                pltpu.VMEM_SHARED((NPAD, D), _f32),
                pltpu.VMEM_SHARED((NPAD, 16), _f32),
                pltpu.VMEM((SUB, CHUNK), jnp.int32),
                pltpu.VMEM((KB, D), _f32),
                pltpu.VMEM((CHUNK, D), _f32),
                pltpu.VMEM((CHUNK, 16), _f32),
                pltpu.VMEM((CHUNK, 16), _f32),
                pltpu.SemaphoreType.DMA,
                pltpu.SemaphoreType.DMA,
            ],
        )(msg, dst2d, z32, z16, ones16)
    return pl.kernel(
        functools.partial(_sc_scatter_body, False),
        out_type=jax.ShapeDtypeStruct((NC, NPAD, D), _f32),
        mesh=mesh,
        compiler_params=pltpu.CompilerParams(use_tc_tiling_on_sc=False),
        scratch_types=[
            pltpu.VMEM_SHARED((NPAD, D), _f32),
            pltpu.VMEM((SUB, CHUNK), jnp.int32),
            pltpu.VMEM((KB, D), _f32),
            pltpu.VMEM((CHUNK, D), _f32),
            pltpu.SemaphoreType.DMA,
        ],
    )(msg, dst2d, z32)


def kernel(x, edge_attr, W0, b0, L1, bn1, L2, bn2, Wroot, cb, Wih, Whh, bih,
           bhh, Wih_s, Whh_s, bih_s, bhh_s, Wl1, bl1, Wl2, bl2, edge_index,
           batch):
    src = edge_index[0].astype(jnp.int32)
    dst = edge_index[1].astype(jnp.int32)
    src2d = jnp.concatenate(
        [src, jnp.zeros((EPAD - E,), jnp.int32)]).reshape(EPAD // CHUNK, CHUNK)
    dst2d = jnp.concatenate(
        [dst, jnp.full((EPAD - E,), N, jnp.int32)]).reshape(
            EPAD // CHUNK, CHUNK)
    hrows = EHALF // CHUNK
    src2d_h = [src2d[i * hrows:(i + 1) * hrows] for i in range(NHALF)]
    dst2d_h = [dst2d[i * hrows:(i + 1) * hrows] for i in range(NHALF)]

    Rm = jnp.repeat(jnp.eye(D, dtype=_f32), D, axis=1)
    Sm = jnp.tile(jnp.eye(D, dtype=_f32), (D, 1))
    Bn2r = bn2.reshape(D, D)

    z32 = jnp.zeros((CHUNK, D), _f32)
    z16 = jnp.zeros((CHUNK, 16), _f32)
    ones16 = jnp.ones((CHUNK, 16), _f32)

    WihT = Wih.T
    WhhT = Whh.T
    WihsT = Wih_s.T
    WhhsT = Whh_s.T
    bias_s = (bih_s + bhh_s).reshape(1, 4 * D)
    bcol = batch.astype(jnp.int32).reshape(N, 1)

    out = _proj(x, W0, b0)
    degs = None
    for layer in range(NCONV):
        xs_h = [_sc_gather(out, src2d_h[i]) for i in range(NHALF)]
        msg_h = [_msg(edge_attr, xs_h[i], L1, bn1, L2, Bn2r, Rm, Sm,
                      i * (EHALF // EB)) for i in range(NHALF)]
        if layer == 0:
            ad_h = [_sc_scatter(msg_h[i], dst2d_h[i], z32, z16, ones16)
                    for i in range(NHALF)]
            aggs = [a for a, _ in ad_h]
            degs = [d for _, d in ad_h]
        else:
            aggs = [_sc_scatter(msg_h[i], dst2d_h[i], z32, with_deg=False)
                    for i in range(NHALF)]
        out = _gru(aggs, degs, out, Wroot, cb, WihT, WhhT, bih, bhh)

    o2 = _s2s(out, bcol, WihsT, WhhsT, bias_s, Wl1, bl1, Wl2, bl2)
    return o2.reshape(-1)


# speedup vs baseline: 1.0278x; 1.0278x over previous
import functools

import jax
import jax.numpy as jnp
from jax import lax
from jax.experimental import pallas as pl
from jax.experimental.pallas import tpu as pltpu
from jax.experimental.pallas import tpu_sc as plsc

N = 10000
E = 160000
DF = 128
DE = 16
H = 128
D = 32
B = 128
NCONV = 2
NS2S = 2

NC = 2
NS = 16
NW = NC * NS
CHUNK = 128
SUB = 10
KB = SUB * CHUNK
CPW = 2
EPW = CPW * KB
EHALF = NW * EPW
NHALF = 2
EPAD = NHALF * EHALF
NPAD = 10240
ZR = NPAD // NS

EB = 2048

_f32 = jnp.float32


def _proj_body(x_ref, w_ref, b_ref, o_ref):
    o_ref[...] = jnp.maximum(
        jnp.dot(x_ref[...], w_ref[...], preferred_element_type=_f32)
        + b_ref[...], 0.0)


def _proj(x, W0, b0):
    return pl.pallas_call(
        _proj_body,
        out_shape=jax.ShapeDtypeStruct((N, D), _f32),
    )(x, W0, b0.reshape(1, D))


def _msg_body(ea_ref, xs_ref, l1_ref, bn1_ref, l2_ref, bn2r_ref, r_ref, s_ref,
              msg_ref):
    hb = EB // 2
    for p in range(2):
        rows = pl.ds(p * hb, hb)
        h1 = jnp.maximum(
            jnp.dot(ea_ref[rows, :], l1_ref[...], preferred_element_type=_f32)
            + bn1_ref[...], 0.0)
        ew = jnp.dot(h1, l2_ref[...], preferred_element_type=_f32)
        xs = xs_ref[rows, :]
        xr = jnp.dot(xs, r_ref[...], preferred_element_type=_f32)
        msg_ref[rows, :] = (
            jnp.dot(ew * xr, s_ref[...], preferred_element_type=_f32)
            + jnp.dot(xs, bn2r_ref[...], preferred_element_type=_f32))


def _msg(ea, xs, L1, bn1, L2, Bn2r, Rm, Sm, blk0):
    grid = (EHALF // EB,)
    nblk = (E - 1) // EB
    full = lambda shape: pl.BlockSpec(shape, lambda i: (0, 0))
    return pl.pallas_call(
        _msg_body,
        grid=grid,
        in_specs=[
            pl.BlockSpec((EB, DE), lambda i: (jnp.minimum(i + blk0, nblk), 0)),
            pl.BlockSpec((EB, D), lambda i: (i, 0)),
            full((DE, H)),
            full((1, H)),
            pl.BlockSpec((H, D * D), lambda i: (0, 0)),
            full((D, D)),
            full((D, D * D)),
            full((D * D, D)),
        ],
        out_specs=pl.BlockSpec((EB, D), lambda i: (i, 0)),
        out_shape=jax.ShapeDtypeStruct((EHALF, D), _f32),
    )(ea, xs, L1, bn1.reshape(1, H), L2, Bn2r, Rm, Sm)


NB = 2000


def _gru_body(*refs):
    agg_refs = refs[0:NHALF]
    deg_refs = refs[NHALF:2 * NHALF]
    (out_ref, wroot_ref, cb_ref, wih_ref, whh_ref, bih_ref, bhh_ref,
     new_ref) = refs[2 * NHALF:]
    d = jnp.maximum(
        sum(r[c, :, 0:1] for r in deg_refs for c in range(NC)), 1.0)
    agg = sum(r[c, :, :] for r in agg_refs for c in range(NC)) / d
    out = out_ref[...]
    m = jnp.maximum(
        agg + jnp.dot(out, wroot_ref[...], preferred_element_type=_f32)
        + cb_ref[...], 0.0)
    gi = jnp.dot(m, wih_ref[...], preferred_element_type=_f32) + bih_ref[...]
    gh = jnp.dot(out, whh_ref[...], preferred_element_type=_f32) + bhh_ref[...]
    r = jax.nn.sigmoid(gi[:, 0:D] + gh[:, 0:D])
    z = jax.nn.sigmoid(gi[:, D:2 * D] + gh[:, D:2 * D])
    n = jnp.tanh(gi[:, 2 * D:3 * D] + r * gh[:, 2 * D:3 * D])
    new_ref[...] = (1.0 - z) * n + z * out


def _gru(aggs, degs, out, Wroot, cb, WihT, WhhT, bih, bhh):
    full = lambda shape: pl.BlockSpec(shape, lambda i: tuple(0 for _ in shape))
    return pl.pallas_call(
        _gru_body,
        grid=(N // NB,),
        in_specs=(
            [pl.BlockSpec((NC, NB, D), lambda i: (0, i, 0))] * NHALF
            + [pl.BlockSpec((NC, NB, 16), lambda i: (0, i, 0))] * NHALF
            + [
                pl.BlockSpec((NB, D), lambda i: (i, 0)),
                full((D, D)),
                full((1, D)),
                full((D, 3 * D)),
                full((D, 3 * D)),
                full((1, 3 * D)),
                full((1, 3 * D)),
            ]),
        out_specs=pl.BlockSpec((NB, D), lambda i: (i, 0)),
        out_shape=jax.ShapeDtypeStruct((N, D), _f32),
    )(*aggs, *degs, out, Wroot, cb.reshape(1, D), WihT, WhhT,
      bih.reshape(1, 3 * D), bhh.reshape(1, 3 * D))


def _s2s_body(out_ref, bcol_ref, wih_ref, whh_ref, bias_ref, wl1_ref, bl1_ref,
              wl2_ref, bl2_ref, o2_ref):
    out = out_ref[...]
    iota_row = lax.broadcasted_iota(jnp.int32, (1, B), 1)
    oh = (bcol_ref[...] == iota_row).astype(_f32)
    hs = jnp.zeros((B, D), _f32)
    cs = jnp.zeros((B, D), _f32)
    q_star = jnp.zeros((B, 2 * D), _f32)
    for _ in range(NS2S):
        gates = (jnp.dot(q_star, wih_ref[...], preferred_element_type=_f32)
                 + jnp.dot(hs, whh_ref[...], preferred_element_type=_f32)
                 + bias_ref[...])
        i_ = jax.nn.sigmoid(gates[:, 0:D])
        f_ = jax.nn.sigmoid(gates[:, D:2 * D])
        g_ = jnp.tanh(gates[:, 2 * D:3 * D])
        o_ = jax.nn.sigmoid(gates[:, 3 * D:4 * D])
        cs = f_ * cs + i_ * g_
        hs = o_ * jnp.tanh(cs)
        qb = jnp.dot(oh, hs, preferred_element_type=_f32)
        e = jnp.sum(out * qb, axis=1, keepdims=True)
        mx = jnp.max(jnp.where(oh > 0.0, e, -1e30), axis=0, keepdims=True)
        mxb = jnp.max(jnp.where(oh > 0.0, mx, -1e30), axis=1, keepdims=True)
        ex = jnp.exp(e - mxb)
        den = jnp.sum(oh * ex, axis=0, keepdims=True)
        denb = jnp.sum(jnp.where(oh > 0.0, den, 0.0), axis=1, keepdims=True)
        a = ex / (denb + 1e-16)
        rvec = lax.dot_general(oh, a * out, (((0,), (0,)), ((), ())),
                               preferred_element_type=_f32)
        q_star = jnp.concatenate([hs, rvec], axis=1)
    o1 = jnp.maximum(
        jnp.dot(q_star, wl1_ref[...], preferred_element_type=_f32)
        + bl1_ref[...], 0.0)
    o2_ref[...] = (jnp.dot(o1, wl2_ref[...], preferred_element_type=_f32)
                   + bl2_ref[...])


def _s2s(out, bcol, WihsT, WhhsT, bias_s, Wl1, bl1, Wl2, bl2):
    return pl.pallas_call(
        _s2s_body,
        out_shape=jax.ShapeDtypeStruct((B, 1), _f32),
    )(out, bcol, WihsT, WhhsT, bias_s, Wl1, bl1.reshape(1, D),
      Wl2, bl2.reshape(1, 1))


def _sc_gather_body(table_hbm, idx_hbm, xs_hbm, idx_v, rows_v, sem):
    c = lax.axis_index("c")
    s = lax.axis_index("s")
    wid = s * NC + c
    row0 = wid * (EPW // CHUNK)

    def body(k, carry):
        r0 = row0 + k * SUB
        pltpu.sync_copy(idx_hbm.at[pl.ds(r0, SUB)], idx_v)
        descs = []
        for j in range(SUB):
            descs.append(pltpu.async_copy(
                table_hbm.at[idx_v.at[j]],
                rows_v.at[pl.ds(j * CHUNK, CHUNK)], sem))
        for dsc in descs:
            dsc.wait()
        pltpu.sync_copy(rows_v, xs_hbm.at[pl.ds(r0 * CHUNK, KB)])
        return carry

    lax.fori_loop(0, CPW, body, 0)


def _sc_gather(out, src2d):
    mesh = plsc.VectorSubcoreMesh(core_axis_name="c", subcore_axis_name="s")
    return pl.kernel(
        _sc_gather_body,
        out_type=jax.ShapeDtypeStruct((EHALF, D), _f32),
        mesh=mesh,
        compiler_params=pltpu.CompilerParams(use_tc_tiling_on_sc=False),
        scratch_types=[
            pltpu.VMEM((SUB, CHUNK), jnp.int32),
            pltpu.VMEM((KB, D), _f32),
            pltpu.SemaphoreType.DMA,
        ],
    )(out, src2d)


ZB = ZR // CHUNK


def _sc_scatter_body(with_deg, *refs):
    if with_deg:
        (msg_hbm, idx_hbm, z32_hbm, z16_hbm, ones_hbm, agg_hbm, deg_hbm,
         aggbuf, degbuf, idx_v, rows_v, zc_v, z16c_v, ones_v, sem,
         semd) = refs
    else:
        (msg_hbm, idx_hbm, z32_hbm, agg_hbm,
         aggbuf, idx_v, rows_v, zc_v, sem) = refs
    c = lax.axis_index("c")
    s = lax.axis_index("s")
    wid = s * NC + c
    row0 = wid * (EPW // CHUNK)

    pltpu.sync_copy(z32_hbm, zc_v)
    for t in range(ZB):
        pltpu.sync_copy(zc_v, aggbuf.at[pl.ds(s * ZR + t * CHUNK, CHUNK)])
    if with_deg:
        pltpu.sync_copy(z16_hbm, z16c_v)
        pltpu.sync_copy(ones_hbm, ones_v)
        for t in range(ZB):
            pltpu.sync_copy(
                z16c_v, degbuf.at[pl.ds(s * ZR + t * CHUNK, CHUNK)])
    plsc.subcore_barrier()

    def body(k, carry):
        r0 = row0 + k * SUB
        pltpu.sync_copy(idx_hbm.at[pl.ds(r0, SUB)], idx_v)
        pltpu.sync_copy(msg_hbm.at[pl.ds(r0 * CHUNK, KB)], rows_v)
        descs = []
        for j in range(SUB):
            descs.append(pltpu.async_copy(
                rows_v.at[pl.ds(j * CHUNK, CHUNK)],
                aggbuf.at[idx_v.at[j]], sem, add=True))
            if with_deg:
                descs.append(pltpu.async_copy(
                    ones_v, degbuf.at[idx_v.at[j]], semd, add=True))
        for dsc in descs:
            dsc.wait()
        return carry

    lax.fori_loop(0, CPW, body, 0)
    plsc.subcore_barrier()

    pltpu.sync_copy(aggbuf.at[pl.ds(s * ZR, ZR)],
                    agg_hbm.at[c, pl.ds(s * ZR, ZR)])
    if with_deg:
        pltpu.sync_copy(degbuf.at[pl.ds(s * ZR, ZR)],
                        deg_hbm.at[c, pl.ds(s * ZR, ZR)])


def _sc_scatter(msg, dst2d, z32, z16=None, ones16=None, with_deg=True):
    mesh = plsc.VectorSubcoreMesh(core_axis_name="c", subcore_axis_name="s")
    if with_deg:
        return pl.kernel(
            functools.partial(_sc_scatter_body, True),
            out_type=[
                jax.ShapeDtypeStruct((NC, NPAD, D), _f32),
                jax.ShapeDtypeStruct((NC, NPAD, 16), _f32),
            ],
            mesh=mesh,
            compiler_params=pltpu.CompilerParams(use_tc_tiling_on_sc=False),
            scratch_types=[
                pltpu.VMEM_SHARED((NPAD, D), _f32),
                pltpu.VMEM_SHARED((NPAD, 16), _f32),
                pltpu.VMEM((SUB, CHUNK), jnp.int32),
                pltpu.VMEM((KB, D), _f32),
                pltpu.VMEM((CHUNK, D), _f32),
                pltpu.VMEM((CHUNK, 16), _f32),
                pltpu.VMEM((CHUNK, 16), _f32),
                pltpu.SemaphoreType.DMA,
                pltpu.SemaphoreType.DMA,
            ],
        )(msg, dst2d, z32, z16, ones16)
    return pl.kernel(
        functools.partial(_sc_scatter_body, False),
        out_type=jax.ShapeDtypeStruct((NC, NPAD, D), _f32),
        mesh=mesh,
        compiler_params=pltpu.CompilerParams(use_tc_tiling_on_sc=False),
        scratch_types=[
            pltpu.VMEM_SHARED((NPAD, D), _f32),
            pltpu.VMEM((SUB, CHUNK), jnp.int32),
            pltpu.VMEM((KB, D), _f32),
            pltpu.VMEM((CHUNK, D), _f32),
            pltpu.SemaphoreType.DMA,
        ],
    )(msg, dst2d, z32)


def kernel(x, edge_attr, W0, b0, L1, bn1, L2, bn2, Wroot, cb, Wih, Whh, bih,
           bhh, Wih_s, Whh_s, bih_s, bhh_s, Wl1, bl1, Wl2, bl2, edge_index,
           batch):
    src = edge_index[0].astype(jnp.int32)
    dst = edge_index[1].astype(jnp.int32)
    src2d = jnp.concatenate(
        [src, jnp.zeros((EPAD - E,), jnp.int32)]).reshape(EPAD // CHUNK, CHUNK)
    dst2d = jnp.concatenate(
        [dst, jnp.full((EPAD - E,), N, jnp.int32)]).reshape(
            EPAD // CHUNK, CHUNK)
    hrows = EHALF // CHUNK
    src2d_h = [src2d[i * hrows:(i + 1) * hrows] for i in range(NHALF)]
    dst2d_h = [dst2d[i * hrows:(i + 1) * hrows] for i in range(NHALF)]

    Rm = jnp.repeat(jnp.eye(D, dtype=_f32), D, axis=1)
    Sm = jnp.tile(jnp.eye(D, dtype=_f32), (D, 1))
    Bn2r = bn2.reshape(D, D)

    z32 = jnp.zeros((CHUNK, D), _f32)
    z16 = jnp.zeros((CHUNK, 16), _f32)
    ones16 = jnp.ones((CHUNK, 16), _f32)

    WihT = Wih.T
    WhhT = Whh.T
    WihsT = Wih_s.T
    WhhsT = Whh_s.T
    bias_s = (bih_s + bhh_s).reshape(1, 4 * D)
    bcol = batch.astype(jnp.int32).reshape(N, 1)

    out = _proj(x, W0, b0)
    degs = None
    for layer in range(NCONV):
        xs_h = [_sc_gather(out, src2d_h[i]) for i in range(NHALF)]
        msg_h = [_msg(edge_attr, xs_h[i], L1, bn1, L2, Bn2r, Rm, Sm,
                      i * (EHALF // EB)) for i in range(NHALF)]
        if layer == 0:
            ad_h = [_sc_scatter(msg_h[i], dst2d_h[i], z32, z16, ones16)
                    for i in range(NHALF)]
            aggs = [a for a, _ in ad_h]
            degs = [d for _, d in ad_h]
        else:
            aggs = [_sc_scatter(msg_h[i], dst2d_h[i], z32, with_deg=False)
                    for i in range(NHALF)]
        out = _gru(aggs, degs, out, Wroot, cb, WihT, WhhT, bih, bhh)

    o2 = _s2s(out, bcol, WihsT, WhhsT, bias_s, Wl1, bl1, Wl2, bl2)
    return o2.reshape(-1)
